# initial kernel scaffold (unmeasured)
import jax
import jax.numpy as jnp
from jax import lax
from jax.experimental import pallas as pl
from jax.experimental.pallas import tpu as pltpu

N_DEV = 8


def kernel(x, w_mat):
    m_per, k = x.shape
    k2, n_per = w_mat.shape
    assert k == k2

    def body(x_ref, w_ref, out_ref, comm_ref, send_sems, recv_sems):
        my = lax.axis_index("i")
        left = lax.rem(my + N_DEV - 1, N_DEV)
        right = lax.rem(my + 1, N_DEV)

        barrier_sem = pltpu.get_barrier_semaphore()
        for nbr in (left, right):
            pl.semaphore_signal(
                barrier_sem, inc=1,
                device_id=(nbr,), device_id_type=pl.DeviceIdType.MESH,
            )
        pl.semaphore_wait(barrier_sem, 2)

        def gemm(src, origin):
            acc = jnp.dot(src, w_ref[...], preferred_element_type=jnp.float32)
            out_ref[pl.ds(origin * m_per, m_per), :] = acc / (
                1.0 + jnp.exp(-acc)
            )

        rdma = pltpu.make_async_remote_copy(
            src_ref=x_ref,
            dst_ref=comm_ref.at[0],
            send_sem=send_sems.at[0],
            recv_sem=recv_sems.at[0],
            device_id=(right,),
            device_id_type=pl.DeviceIdType.MESH,
        )
        rdma.start()
        gemm(x_ref[...], my)
        rdma.wait()

        for h in range(1, N_DEV - 1):
            src_slot = (h - 1) % 2
            dst_slot = h % 2
            rdma = pltpu.make_async_remote_copy(
                src_ref=comm_ref.at[src_slot],
                dst_ref=comm_ref.at[dst_slot],
                send_sem=send_sems.at[h],
                recv_sem=recv_sems.at[h],
                device_id=(right,),
                device_id_type=pl.DeviceIdType.MESH,
            )
            rdma.start()
            gemm(comm_ref[src_slot], lax.rem(my + N_DEV - h, N_DEV))
            rdma.wait()

        gemm(comm_ref[(N_DEV - 2) % 2], right)

    return pl.pallas_call(
        body,
        out_shape=jax.ShapeDtypeStruct((N_DEV * m_per, n_per), jnp.float32),
        in_specs=[
            pl.BlockSpec(memory_space=pltpu.VMEM),
            pl.BlockSpec(memory_space=pltpu.VMEM),
        ],
        out_specs=pl.BlockSpec(memory_space=pltpu.VMEM),
        scratch_shapes=[
            pltpu.VMEM((2, m_per, k), jnp.float32),
            pltpu.SemaphoreType.DMA((N_DEV - 1,)),
            pltpu.SemaphoreType.DMA((N_DEV - 1,)),
        ],
        compiler_params=pltpu.CompilerParams(collective_id=0),
    )(x, w_mat)


# baseline (device time: 677714 ns/iter reference)
import jax
import jax.numpy as jnp
from jax import lax
from jax.experimental import pallas as pl
from jax.experimental.pallas import tpu as pltpu

N_DEV = 8


def kernel(x, w_mat):
    m_per, k = x.shape
    k2, n_per = w_mat.shape
    assert k == k2

    def body(x_ref, w_ref, out_ref, comm_ref, send_sems, recv_sems):
        my = lax.axis_index("i")
        left = lax.rem(my + N_DEV - 1, N_DEV)
        right = lax.rem(my + 1, N_DEV)

        barrier_sem = pltpu.get_barrier_semaphore()
        for nbr in (left, right):
            pl.semaphore_signal(
                barrier_sem, inc=1,
                device_id=(nbr,), device_id_type=pl.DeviceIdType.MESH,
            )
        pl.semaphore_wait(barrier_sem, 2)

        def gemm(src, origin):
            acc = jnp.dot(src, w_ref[...], preferred_element_type=jnp.float32)
            out_ref[pl.ds(origin * m_per, m_per), :] = acc / (
                1.0 + jnp.exp(-acc)
            )

        rdma = pltpu.make_async_remote_copy(
            src_ref=x_ref,
            dst_ref=comm_ref.at[0],
            send_sem=send_sems.at[0],
            recv_sem=recv_sems.at[0],
            device_id=(right,),
            device_id_type=pl.DeviceIdType.MESH,
        )
        rdma.start()
        gemm(x_ref[...], my)
        rdma.wait()

        for h in range(1, N_DEV - 1):
            src_slot = (h - 1) % 2
            dst_slot = h % 2
            rdma = pltpu.make_async_remote_copy(
                src_ref=comm_ref.at[src_slot],
                dst_ref=comm_ref.at[dst_slot],
                send_sem=send_sems.at[h],
                recv_sem=recv_sems.at[h],
                device_id=(right,),
                device_id_type=pl.DeviceIdType.MESH,
            )
            rdma.start()
            gemm(comm_ref[src_slot], lax.rem(my + N_DEV - h, N_DEV))
            rdma.wait()

        gemm(comm_ref[(N_DEV - 2) % 2], right)

    return pl.pallas_call(
        body,
        out_shape=jax.ShapeDtypeStruct((N_DEV * m_per, n_per), jnp.float32),
        in_specs=[
            pl.BlockSpec(memory_space=pltpu.VMEM),
            pl.BlockSpec(memory_space=pltpu.VMEM),
        ],
        out_specs=pl.BlockSpec(memory_space=pltpu.VMEM),
        scratch_shapes=[
            pltpu.VMEM((2, m_per, k), jnp.float32),
            pltpu.SemaphoreType.DMA((N_DEV - 1,)),
            pltpu.SemaphoreType.DMA((N_DEV - 1,)),
        ],
        compiler_params=pltpu.CompilerParams(
            collective_id=0,
            vmem_limit_bytes=100 * 1024 * 1024,
        ),
    )(x, w_mat)


# device time: 400016 ns/iter; 1.6942x vs baseline; 1.6942x over previous
import jax
import jax.numpy as jnp
from jax import lax
from jax.experimental import pallas as pl
from jax.experimental.pallas import tpu as pltpu

N_DEV = 8
F_HOPS = 4
B_HOPS = 3


def kernel(x, w_mat):
    m_per, k = x.shape
    k2, n_per = w_mat.shape
    assert k == k2

    def body(
        x_ref, w_ref, out_ref, fcomm, bcomm, stage,
        fsend, frecv, bsend, brecv, copy_sem,
    ):
        my = lax.axis_index("i")
        left = lax.rem(my + N_DEV - 1, N_DEV)
        right = lax.rem(my + 1, N_DEV)

        barrier_sem = pltpu.get_barrier_semaphore()
        for nbr in (left, right):
            pl.semaphore_signal(
                barrier_sem, inc=1,
                device_id=(nbr,), device_id_type=pl.DeviceIdType.MESH,
            )
        pl.semaphore_wait(barrier_sem, 2)

        def gemm(src, origin):
            acc = jnp.dot(src, w_ref[...], preferred_element_type=jnp.float32)
            stage[...] = acc / (1.0 + jnp.exp(-acc))
            cp = pltpu.make_async_copy(
                stage,
                out_ref.at[pl.ds(origin * m_per, m_per), :],
                copy_sem,
            )
            cp.start()
            cp.wait()

        def fwd_rdma(h):
            return pltpu.make_async_remote_copy(
                src_ref=x_ref if h == 0 else fcomm.at[(h - 1) % 2],
                dst_ref=fcomm.at[h % 2],
                send_sem=fsend.at[h],
                recv_sem=frecv.at[h],
                device_id=(right,),
                device_id_type=pl.DeviceIdType.MESH,
            )

        def bwd_rdma(h):
            return pltpu.make_async_remote_copy(
                src_ref=x_ref if h == 0 else bcomm.at[(h - 1) % 2],
                dst_ref=bcomm.at[h % 2],
                send_sem=bsend.at[h],
                recv_sem=brecv.at[h],
                device_id=(left,),
                device_id_type=pl.DeviceIdType.MESH,
            )

        f = fwd_rdma(0)
        b = bwd_rdma(0)
        f.start()
        b.start()
        gemm(x_ref[...], my)
        f.wait()
        b.wait()

        for h in range(1, F_HOPS):
            f = fwd_rdma(h)
            f.start()
            if h < B_HOPS:
                b = bwd_rdma(h)
                b.start()
            gemm(fcomm[(h - 1) % 2], lax.rem(my + N_DEV - h, N_DEV))
            gemm(bcomm[(h - 1) % 2], lax.rem(my + h, N_DEV))
            f.wait()
            if h < B_HOPS:
                b.wait()

        gemm(fcomm[(F_HOPS - 1) % 2], lax.rem(my + N_DEV - F_HOPS, N_DEV))

    return pl.pallas_call(
        body,
        out_shape=jax.ShapeDtypeStruct((N_DEV * m_per, n_per), jnp.float32),
        in_specs=[
            pl.BlockSpec(memory_space=pltpu.VMEM),
            pl.BlockSpec(memory_space=pltpu.VMEM),
        ],
        out_specs=pl.BlockSpec(memory_space=pl.ANY),
        scratch_shapes=[
            pltpu.VMEM((2, m_per, k), jnp.float32),
            pltpu.VMEM((2, m_per, k), jnp.float32),
            pltpu.VMEM((m_per, n_per), jnp.float32),
            pltpu.SemaphoreType.DMA((F_HOPS,)),
            pltpu.SemaphoreType.DMA((F_HOPS,)),
            pltpu.SemaphoreType.DMA((B_HOPS,)),
            pltpu.SemaphoreType.DMA((B_HOPS,)),
            pltpu.SemaphoreType.DMA,
        ],
        compiler_params=pltpu.CompilerParams(
            collective_id=0,
            vmem_limit_bytes=128 * 1024 * 1024,
        ),
    )(x, w_mat)


# device time: 261087 ns/iter; 2.5957x vs baseline; 1.5321x over previous
import jax
import jax.numpy as jnp
from jax import lax
from jax.experimental import pallas as pl
from jax.experimental.pallas import tpu as pltpu

N_DEV = 8
MASKS = (1, 3, 4)
ORDERS = ((1, 3, 4), (3, 4, 1), (4, 1, 3))
ROW_OFF = (0, 176, 344)
ROW_SZ = (176, 168, 168)


def kernel(x, w_mat):
    m_per, k = x.shape
    k2, n_per = w_mat.shape
    assert k == k2

    def body(
        x_ref, w_ref, out_ref,
        r2l0, r2l1, r2l2,
        r0l0, r0l1, r0l2,
        r1l0, r1l1, r1l2,
        gbuf, stage,
        sems0, sems1, sems2,
        remr0, remr1, remr2,
        csem, osem,
    ):
        r0l = (r0l0, r0l1, r0l2)
        r1l = (r1l0, r1l1, r1l2)
        r2l = (r2l0, r2l1, r2l2)
        ssems = (sems0, sems1, sems2)
        rsems = (remr0, remr1, remr2)

        my = lax.axis_index("i")

        barrier_sem = pltpu.get_barrier_semaphore()
        for mask in MASKS:
            pl.semaphore_signal(
                barrier_sem, inc=1,
                device_id=(my ^ mask,), device_id_type=pl.DeviceIdType.MESH,
            )
        pl.semaphore_wait(barrier_sem, len(MASKS))

        def gemm_out(src, origin, row_off, rows):
            acc = jnp.dot(src, w_ref[...], preferred_element_type=jnp.float32)
            stage[pl.ds(0, rows), :] = acc / (1.0 + jnp.exp(-acc))
            cp = pltpu.make_async_copy(
                stage.at[pl.ds(0, rows), :],
                out_ref.at[pl.ds(origin * m_per + row_off, rows), :],
                osem,
            )
            cp.start()
            cp.wait()

        def exch(t, sub, src_ref, dst_ref, partner):
            return pltpu.make_async_remote_copy(
                src_ref=src_ref,
                dst_ref=dst_ref,
                send_sem=ssems[t].at[sub],
                recv_sem=rsems[t].at[sub],
                device_id=(partner,),
                device_id_type=pl.DeviceIdType.MESH,
            )

        def xslice(t):
            return x_ref.at[pl.ds(ROW_OFF[t], ROW_SZ[t]), :]

        started = []

        for t in range(3):
            c = exch(t, 0, xslice(t), r0l[t], my ^ ORDERS[t][0])
            c.start()
            started.append(c)
        gemm_out(x_ref[...], my, 0, m_per)
        for t in range(3):
            started[t].wait_recv()

        r1 = []
        for t in range(3):
            p = my ^ ORDERS[t][1]
            a = exch(t, 1, xslice(t), r1l[t].at[0], p)
            b = exch(t, 2, r0l[t], r1l[t].at[1], p)
            a.start()
            b.start()
            started += [a, b]
            r1.append((a, b))
        for t in range(3):
            gemm_out(r0l[t][...], my ^ ORDERS[t][0], ROW_OFF[t], ROW_SZ[t])
        for a, b in r1:
            a.wait_recv()
            b.wait_recv()

        r2 = []
        for t in range(3):
            p = my ^ ORDERS[t][2]
            subs = [
                exch(t, 3, xslice(t), r2l[t].at[0], p),
                exch(t, 4, r0l[t], r2l[t].at[1], p),
                exch(t, 5, r1l[t].at[0], r2l[t].at[2], p),
                exch(t, 6, r1l[t].at[1], r2l[t].at[3], p),
            ]
            for c in subs:
                c.start()
            started += subs
            r2.append(subs)
        for t in range(3):
            m0, m1 = ORDERS[t][0], ORDERS[t][1]
            gemm_out(r1l[t][0], my ^ m1, ROW_OFF[t], ROW_SZ[t])
            gemm_out(r1l[t][1], my ^ m1 ^ m0, ROW_OFF[t], ROW_SZ[t])

        for j in range(4):
            for t in range(3):
                r2[t][j].wait_recv()
                m0, m1, m2 = ORDERS[t]
                origin = my ^ m2 ^ (0, m0, m1, m1 ^ m0)[j]
                cp = pltpu.make_async_copy(
                    r2l[t].at[j], gbuf.at[pl.ds(0, ROW_SZ[t]), :], csem
                )
                cp.start()
                cp.wait()
                gemm_out(gbuf[pl.ds(0, ROW_SZ[t]), :], origin,
                         ROW_OFF[t], ROW_SZ[t])

        for c in started:
            c.wait_send()

    sdma = pltpu.SemaphoreType.DMA
    out, _, _, _ = pl.pallas_call(
        body,
        out_shape=[
            jax.ShapeDtypeStruct((N_DEV * m_per, n_per), jnp.float32),
            jax.ShapeDtypeStruct((4, ROW_SZ[0], k), jnp.float32),
            jax.ShapeDtypeStruct((4, ROW_SZ[1], k), jnp.float32),
            jax.ShapeDtypeStruct((4, ROW_SZ[2], k), jnp.float32),
        ],
        in_specs=[
            pl.BlockSpec(memory_space=pltpu.VMEM),
            pl.BlockSpec(memory_space=pltpu.VMEM),
        ],
        out_specs=[
            pl.BlockSpec(memory_space=pl.ANY),
            pl.BlockSpec(memory_space=pl.ANY),
            pl.BlockSpec(memory_space=pl.ANY),
            pl.BlockSpec(memory_space=pl.ANY),
        ],
        scratch_shapes=[
            pltpu.VMEM((ROW_SZ[0], k), jnp.float32),
            pltpu.VMEM((ROW_SZ[1], k), jnp.float32),
            pltpu.VMEM((ROW_SZ[2], k), jnp.float32),
            pltpu.VMEM((2, ROW_SZ[0], k), jnp.float32),
            pltpu.VMEM((2, ROW_SZ[1], k), jnp.float32),
            pltpu.VMEM((2, ROW_SZ[2], k), jnp.float32),
            pltpu.VMEM((ROW_SZ[0], k), jnp.float32),
            pltpu.VMEM((m_per, n_per), jnp.float32),
            sdma((7,)), sdma((7,)), sdma((7,)),
            sdma((7,)), sdma((7,)), sdma((7,)),
            sdma,
            sdma,
        ],
        compiler_params=pltpu.CompilerParams(
            collective_id=0,
            vmem_limit_bytes=128 * 1024 * 1024,
        ),
    )(x, w_mat)
    return out


# device time: 248837 ns/iter; 2.7235x vs baseline; 1.0492x over previous
import jax
import jax.numpy as jnp
from jax import lax
from jax.experimental import pallas as pl
from jax.experimental.pallas import tpu as pltpu

N_DEV = 8
MASKS = (1, 3, 4)
ORDERS = ((1, 3, 4), (3, 4, 1), (4, 1, 3))
ROW_OFF = (0, 176, 344)
ROW_SZ = (176, 168, 168)


def kernel(x, w_mat):
    m_per, k = x.shape
    k2, n_per = w_mat.shape
    assert k == k2

    def body(
        x_ref, w_ref, out_ref,
        r2l0, r2l1, r2l2,
        r0l0, r0l1, r0l2,
        r1l0, r1l1, r1l2,
        gbuf, stage,
        sems0, sems1, sems2,
        remr0, remr1, remr2,
        csems, osems,
    ):
        r0l = (r0l0, r0l1, r0l2)
        r1l = (r1l0, r1l1, r1l2)
        r2l = (r2l0, r2l1, r2l2)
        ssems = (sems0, sems1, sems2)
        rsems = (remr0, remr1, remr2)

        my = lax.axis_index("i")

        barrier_sem = pltpu.get_barrier_semaphore()
        for mask in MASKS:
            pl.semaphore_signal(
                barrier_sem, inc=1,
                device_id=(my ^ mask,), device_id_type=pl.DeviceIdType.MESH,
            )
        pl.semaphore_wait(barrier_sem, len(MASKS))

        pending_out = [None, None]
        slot_ctr = [0]

        def gemm_out(src, origin, row_off, rows):
            s = slot_ctr[0] % 2
            slot_ctr[0] += 1
            acc = jnp.dot(src, w_ref[...], preferred_element_type=jnp.float32)
            if pending_out[s] is not None:
                pending_out[s].wait()
            stage[s, pl.ds(0, rows), :] = acc / (1.0 + jnp.exp(-acc))
            cp = pltpu.make_async_copy(
                stage.at[s, pl.ds(0, rows), :],
                out_ref.at[pl.ds(origin * m_per + row_off, rows), :],
                osems.at[s],
            )
            cp.start()
            pending_out[s] = cp

        def exch(t, sub, src_ref, dst_ref, partner):
            return pltpu.make_async_remote_copy(
                src_ref=src_ref,
                dst_ref=dst_ref,
                send_sem=ssems[t].at[sub],
                recv_sem=rsems[t].at[sub],
                device_id=(partner,),
                device_id_type=pl.DeviceIdType.MESH,
            )

        def xslice(t):
            return x_ref.at[pl.ds(ROW_OFF[t], ROW_SZ[t]), :]

        started = []

        def start(c):
            c.start()
            started.append(c)
            return c

        r0 = [start(exch(t, 0, xslice(t), r0l[t], my ^ ORDERS[t][0]))
              for t in range(3)]
        r1a = [start(exch(t, 1, xslice(t), r1l[t].at[0], my ^ ORDERS[t][1]))
               for t in range(3)]
        r2sub = [[start(exch(t, 3, xslice(t), r2l[t].at[0], my ^ ORDERS[t][2]))]
                 for t in range(3)]

        gemm_out(x_ref[...], my, 0, m_per)

        r1b = []
        for t in range(3):
            r0[t].wait_recv()
            r1b.append(
                start(exch(t, 2, r0l[t], r1l[t].at[1], my ^ ORDERS[t][1]))
            )
            r2sub[t].append(
                start(exch(t, 4, r0l[t], r2l[t].at[1], my ^ ORDERS[t][2]))
            )
        for t in range(3):
            gemm_out(r0l[t][...], my ^ ORDERS[t][0], ROW_OFF[t], ROW_SZ[t])

        for t in range(3):
            r1a[t].wait_recv()
            r1b[t].wait_recv()
            p2 = my ^ ORDERS[t][2]
            r2sub[t].append(
                start(exch(t, 5, r1l[t].at[0], r2l[t].at[2], p2))
            )
            r2sub[t].append(
                start(exch(t, 6, r1l[t].at[1], r2l[t].at[3], p2))
            )
        for t in range(3):
            m0, m1 = ORDERS[t][0], ORDERS[t][1]
            gemm_out(r1l[t][0], my ^ m1, ROW_OFF[t], ROW_SZ[t])
            gemm_out(r1l[t][1], my ^ m1 ^ m0, ROW_OFF[t], ROW_SZ[t])

        work = []
        for j in range(4):
            for t in range(3):
                m0, m1, m2 = ORDERS[t]
                origin = my ^ m2 ^ (0, m0, m1, m1 ^ m0)[j]
                work.append((j, t, origin))
        prev = None
        for i, (j, t, origin) in enumerate(work):
            r2sub[t][j].wait_recv()
            s = i % 2
            cp = pltpu.make_async_copy(
                r2l[t].at[j], gbuf.at[s, pl.ds(0, ROW_SZ[t]), :], csems.at[s]
            )
            cp.start()
            if prev is not None:
                pcp, pt, porigin, ps = prev
                pcp.wait()
                gemm_out(gbuf[ps, pl.ds(0, ROW_SZ[pt]), :], porigin,
                         ROW_OFF[pt], ROW_SZ[pt])
            prev = (cp, t, origin, s)
        pcp, pt, porigin, ps = prev
        pcp.wait()
        gemm_out(gbuf[ps, pl.ds(0, ROW_SZ[pt]), :], porigin,
                 ROW_OFF[pt], ROW_SZ[pt])

        for c in started:
            c.wait_send()
        for p in pending_out:
            if p is not None:
                p.wait()

    sdma = pltpu.SemaphoreType.DMA
    out, _, _, _ = pl.pallas_call(
        body,
        out_shape=[
            jax.ShapeDtypeStruct((N_DEV * m_per, n_per), jnp.float32),
            jax.ShapeDtypeStruct((4, ROW_SZ[0], k), jnp.float32),
            jax.ShapeDtypeStruct((4, ROW_SZ[1], k), jnp.float32),
            jax.ShapeDtypeStruct((4, ROW_SZ[2], k), jnp.float32),
        ],
        in_specs=[
            pl.BlockSpec(memory_space=pltpu.VMEM),
            pl.BlockSpec(memory_space=pltpu.VMEM),
        ],
        out_specs=[
            pl.BlockSpec(memory_space=pl.ANY),
            pl.BlockSpec(memory_space=pl.ANY),
            pl.BlockSpec(memory_space=pl.ANY),
            pl.BlockSpec(memory_space=pl.ANY),
        ],
        scratch_shapes=[
            pltpu.VMEM((ROW_SZ[0], k), jnp.float32),
            pltpu.VMEM((ROW_SZ[1], k), jnp.float32),
            pltpu.VMEM((ROW_SZ[2], k), jnp.float32),
            pltpu.VMEM((2, ROW_SZ[0], k), jnp.float32),
            pltpu.VMEM((2, ROW_SZ[1], k), jnp.float32),
            pltpu.VMEM((2, ROW_SZ[2], k), jnp.float32),
            pltpu.VMEM((2, ROW_SZ[0], k), jnp.float32),
            pltpu.VMEM((2, m_per, n_per), jnp.float32),
            sdma((7,)), sdma((7,)), sdma((7,)),
            sdma((7,)), sdma((7,)), sdma((7,)),
            sdma((2,)),
            sdma((2,)),
        ],
        compiler_params=pltpu.CompilerParams(
            collective_id=0,
            vmem_limit_bytes=128 * 1024 * 1024,
        ),
    )(x, w_mat)
    return out


# device time: 156344 ns/iter; 4.3348x vs baseline; 1.5916x over previous
import jax
import jax.numpy as jnp
from jax import lax
from jax.experimental import pallas as pl
from jax.experimental.pallas import tpu as pltpu

N_DEV = 8
MASKS = (1, 3, 4)
ORDERS = ((1, 3, 4), (3, 4, 1), (4, 1, 3))
ROW_OFF = (0, 176, 344)
ROW_SZ = (176, 168, 168)


def kernel(x, w_mat):
    m_per, k = x.shape
    k2, n_per = w_mat.shape
    assert k == k2

    def body(
        x_ref, w_ref, out_ref,
        r2l0, r2l1, r2l2,
        r0l0, r0l1, r0l2,
        r1l0, r1l1, r1l2,
        gbuf,
        sems0, sems1, sems2,
        remr0, remr1, remr2,
        csems,
    ):
        r0l = (r0l0, r0l1, r0l2)
        r1l = (r1l0, r1l1, r1l2)
        r2l = (r2l0, r2l1, r2l2)
        ssems = (sems0, sems1, sems2)
        rsems = (remr0, remr1, remr2)

        my = lax.axis_index("i")

        barrier_sem = pltpu.get_barrier_semaphore()
        for mask in MASKS:
            pl.semaphore_signal(
                barrier_sem, inc=1,
                device_id=(my ^ mask,), device_id_type=pl.DeviceIdType.MESH,
            )
        pl.semaphore_wait(barrier_sem, len(MASKS))

        def gemm_out(src, origin, row_off, rows):
            acc = jnp.dot(src, w_ref[...], preferred_element_type=jnp.float32)
            out_ref[pl.ds(origin * m_per + row_off, rows), :] = acc / (
                1.0 + jnp.exp(-acc)
            )

        def exch(t, sub, src_ref, dst_ref, partner):
            return pltpu.make_async_remote_copy(
                src_ref=src_ref,
                dst_ref=dst_ref,
                send_sem=ssems[t].at[sub],
                recv_sem=rsems[t].at[sub],
                device_id=(partner,),
                device_id_type=pl.DeviceIdType.MESH,
            )

        def xslice(t):
            return x_ref.at[pl.ds(ROW_OFF[t], ROW_SZ[t]), :]

        started = []

        def start(c):
            c.start()
            started.append(c)
            return c

        r0 = [start(exch(t, 0, xslice(t), r0l[t], my ^ ORDERS[t][0]))
              for t in range(3)]
        r1a = [start(exch(t, 1, xslice(t), r1l[t].at[0], my ^ ORDERS[t][1]))
               for t in range(3)]
        r2sub = [[start(exch(t, 3, xslice(t), r2l[t].at[0], my ^ ORDERS[t][2]))]
                 for t in range(3)]

        gemm_out(x_ref[...], my, 0, m_per)

        r1b = []
        for t in range(3):
            r0[t].wait_recv()
            r1b.append(
                start(exch(t, 2, r0l[t], r1l[t].at[1], my ^ ORDERS[t][1]))
            )
            r2sub[t].append(
                start(exch(t, 4, r0l[t], r2l[t].at[1], my ^ ORDERS[t][2]))
            )
        for t in range(3):
            gemm_out(r0l[t][...], my ^ ORDERS[t][0], ROW_OFF[t], ROW_SZ[t])

        for t in range(3):
            r1a[t].wait_recv()
            r1b[t].wait_recv()
            p2 = my ^ ORDERS[t][2]
            r2sub[t].append(
                start(exch(t, 5, r1l[t].at[0], r2l[t].at[2], p2))
            )
            r2sub[t].append(
                start(exch(t, 6, r1l[t].at[1], r2l[t].at[3], p2))
            )
        for t in range(3):
            m0, m1 = ORDERS[t][0], ORDERS[t][1]
            gemm_out(r1l[t][0], my ^ m1, ROW_OFF[t], ROW_SZ[t])
            gemm_out(r1l[t][1], my ^ m1 ^ m0, ROW_OFF[t], ROW_SZ[t])

        work = []
        for j in range(4):
            for t in range(3):
                m0, m1, m2 = ORDERS[t]
                origin = my ^ m2 ^ (0, m0, m1, m1 ^ m0)[j]
                work.append((j, t, origin))
        prev = None
        for i, (j, t, origin) in enumerate(work):
            r2sub[t][j].wait_recv()
            s = i % 2
            cp = pltpu.make_async_copy(
                r2l[t].at[j], gbuf.at[s, pl.ds(0, ROW_SZ[t]), :], csems.at[s]
            )
            cp.start()
            if prev is not None:
                pcp, pt, porigin, ps = prev
                pcp.wait()
                gemm_out(gbuf[ps, pl.ds(0, ROW_SZ[pt]), :], porigin,
                         ROW_OFF[pt], ROW_SZ[pt])
            prev = (cp, t, origin, s)
        pcp, pt, porigin, ps = prev
        pcp.wait()
        gemm_out(gbuf[ps, pl.ds(0, ROW_SZ[pt]), :], porigin,
                 ROW_OFF[pt], ROW_SZ[pt])

        for c in started:
            c.wait_send()

    bf16 = jnp.bfloat16
    sdma = pltpu.SemaphoreType.DMA
    out, _, _, _ = pl.pallas_call(
        body,
        out_shape=[
            jax.ShapeDtypeStruct((N_DEV * m_per, n_per), jnp.float32),
            jax.ShapeDtypeStruct((4, ROW_SZ[0], k), bf16),
            jax.ShapeDtypeStruct((4, ROW_SZ[1], k), bf16),
            jax.ShapeDtypeStruct((4, ROW_SZ[2], k), bf16),
        ],
        in_specs=[
            pl.BlockSpec(memory_space=pltpu.VMEM),
            pl.BlockSpec(memory_space=pltpu.VMEM),
        ],
        out_specs=[
            pl.BlockSpec(memory_space=pltpu.VMEM),
            pl.BlockSpec(memory_space=pl.ANY),
            pl.BlockSpec(memory_space=pl.ANY),
            pl.BlockSpec(memory_space=pl.ANY),
        ],
        scratch_shapes=[
            pltpu.VMEM((ROW_SZ[0], k), bf16),
            pltpu.VMEM((ROW_SZ[1], k), bf16),
            pltpu.VMEM((ROW_SZ[2], k), bf16),
            pltpu.VMEM((2, ROW_SZ[0], k), bf16),
            pltpu.VMEM((2, ROW_SZ[1], k), bf16),
            pltpu.VMEM((2, ROW_SZ[2], k), bf16),
            pltpu.VMEM((2, ROW_SZ[0], k), bf16),
            sdma((7,)), sdma((7,)), sdma((7,)),
            sdma((7,)), sdma((7,)), sdma((7,)),
            sdma((2,)),
        ],
        compiler_params=pltpu.CompilerParams(
            collective_id=0,
            vmem_limit_bytes=128 * 1024 * 1024,
        ),
    )(x.astype(bf16), w_mat.astype(bf16))
    return out
